# baseline (speedup 1.0000x reference)
import functools

import jax
import jax.numpy as jnp
from jax import lax
from jax.experimental import pallas as pl
from jax.experimental.pallas import tpu as pltpu
from jax.experimental.pallas import tpu_sc as plsc

N_NODES = 100000
DIM = 128
NROWS = 65
NC, NS = 2, 16
NW = NC * NS
CHUNK = 400
NCHUNKS = N_NODES // CHUNK
MAXK = (NCHUNKS + NW - 1) // NW
GROUPS = CHUNK // 16


def _make_sc_gather():
    mesh = plsc.VectorSubcoreMesh(core_axis_name="c", subcore_axis_name="s")

    @functools.partial(
        pl.kernel,
        out_type=jax.ShapeDtypeStruct((N_NODES * DIM,), jnp.float32),
        mesh=mesh,
        compiler_params=pltpu.CompilerParams(needs_layout_passes=False),
        scratch_types=[
            pltpu.VMEM((NROWS * DIM,), jnp.float32),
            pltpu.VMEM((CHUNK,), jnp.int32),
            pltpu.VMEM((CHUNK * DIM,), jnp.float32),
            pltpu.VMEM((CHUNK * DIM,), jnp.float32),
            pltpu.SemaphoreType.DMA,
            pltpu.SemaphoreType.DMA,
        ],
    )
    def sc_gather(deg_hbm, table_hbm, out_hbm,
                  table_v, idx_v, rows0, rows1, sem0, sem1):
        wid = lax.axis_index("s") * NC + lax.axis_index("c")
        nk = (NCHUNKS - wid + NW - 1) // NW
        rows, sems = (rows0, rows1), (sem0, sem1)

        pltpu.sync_copy(table_hbm, table_v)

        lane = lax.iota(jnp.int32, 16)
        rowoff = lane * DIM

        def process(k, b):
            base = (wid + k * NW) * CHUNK

            pl.when(k >= 2)(lambda: pltpu.make_async_copy(
                rows[b], out_hbm.at[pl.ds(0, CHUNK * DIM)], sems[b]).wait())

            pltpu.sync_copy(deg_hbm.at[pl.ds(base, CHUNK)], idx_v)

            def group_body(g, _):
                d_vec = idx_v[pl.ds(g * 16, 16)]
                src0 = d_vec * DIM
                dst0 = rowoff + g * (16 * DIM)
                for j in range(DIM):
                    v = plsc.load_gather(table_v, [src0 + j])
                    plsc.store_scatter(rows[b], [dst0 + j], v)
                return 0

            lax.fori_loop(0, GROUPS, group_body, 0)

            pltpu.async_copy(
                rows[b], out_hbm.at[pl.ds(base * DIM, CHUNK * DIM)], sems[b])

        def outer(i, _):
            for b in range(2):
                k = i * 2 + b
                pl.when(k < nk)(lambda k=k, b=b: process(k, b))
            return 0

        lax.fori_loop(0, (MAXK + 1) // 2, outer, 0)

        for b in range(2):
            pltpu.make_async_copy(
                rows[b], out_hbm.at[pl.ds(0, CHUNK * DIM)], sems[b]).wait()

    return sc_gather


_sc_gather = _make_sc_gather()


def kernel(degrees, table):
    out = _sc_gather(degrees.astype(jnp.int32), table.reshape(-1))
    return out.reshape(N_NODES, DIM)


# speedup vs baseline: 2.3382x; 2.3382x over previous
import functools

import jax
import jax.numpy as jnp
from jax import lax
from jax.experimental import pallas as pl
from jax.experimental.pallas import tpu as pltpu
from jax.experimental.pallas import tpu_sc as plsc

N_NODES = 100000
DIM = 128
NROWS = 65
NC, NS = 2, 16
NW = NC * NS
CHUNK = 400
NCHUNKS = N_NODES // CHUNK
MAXK = (NCHUNKS + NW - 1) // NW
GROUPS = CHUNK // 16


def _make_sc_gather():
    mesh = plsc.VectorSubcoreMesh(core_axis_name="c", subcore_axis_name="s")

    @functools.partial(
        pl.kernel,
        out_type=jax.ShapeDtypeStruct((N_NODES * DIM,), jnp.float32),
        mesh=mesh,
        compiler_params=pltpu.CompilerParams(needs_layout_passes=False),
        scratch_types=[
            pltpu.VMEM((NROWS * DIM,), jnp.float32),
            pltpu.VMEM((CHUNK,), jnp.int32),
            pltpu.VMEM((CHUNK * DIM,), jnp.float32),
            pltpu.VMEM((CHUNK * DIM,), jnp.float32),
            pltpu.SemaphoreType.DMA,
            pltpu.SemaphoreType.DMA,
        ],
    )
    def sc_gather(deg_hbm, table_hbm, out_hbm,
                  table_v, idx_v, rows0, rows1, sem0, sem1):
        wid = lax.axis_index("s") * NC + lax.axis_index("c")
        nk = (NCHUNKS - wid + NW - 1) // NW
        rows, sems = (rows0, rows1), (sem0, sem1)

        pltpu.sync_copy(table_hbm, table_v)

        lane = lax.iota(jnp.int32, 16)
        rowoff = lane * DIM

        def process(k, b):
            base = (wid + k * NW) * CHUNK

            pl.when(k >= 2)(lambda: pltpu.make_async_copy(
                rows[b], out_hbm.at[pl.ds(0, CHUNK * DIM)], sems[b]).wait())

            pltpu.sync_copy(deg_hbm.at[pl.ds(base, CHUNK)], idx_v)

            def group_body(g, _):
                d_vec = idx_v[pl.ds(g * 16, 16)]
                src0 = d_vec * DIM
                dst0 = rowoff + g * (16 * DIM)

                @plsc.parallel_loop(0, DIM, unroll=8)
                def _(j):
                    v = plsc.load_gather(table_v, [src0 + j])
                    plsc.store_scatter(rows[b], [dst0 + j], v)

                return 0

            lax.fori_loop(0, GROUPS, group_body, 0)

            pltpu.async_copy(
                rows[b], out_hbm.at[pl.ds(base * DIM, CHUNK * DIM)], sems[b])

        def outer(i, _):
            for b in range(2):
                k = i * 2 + b
                pl.when(k < nk)(lambda k=k, b=b: process(k, b))
            return 0

        lax.fori_loop(0, (MAXK + 1) // 2, outer, 0)

        for b in range(2):
            pltpu.make_async_copy(
                rows[b], out_hbm.at[pl.ds(0, CHUNK * DIM)], sems[b]).wait()

    return sc_gather


_sc_gather = _make_sc_gather()


def kernel(degrees, table):
    out = _sc_gather(degrees.astype(jnp.int32), table.reshape(-1))
    return out.reshape(N_NODES, DIM)
